# Initial kernel scaffold; baseline (speedup 1.0000x reference)
import jax
import jax.numpy as jnp
from jax.experimental import pallas as pl


def kernel(g):
    raise NotImplementedError("write your pallas kernel here")


# speedup vs baseline: 92.0782x; 92.0782x over previous
import jax
import jax.numpy as jnp
from jax.experimental import pallas as pl
from jax.experimental.pallas import tpu as pltpu


def _ktakes_kernel(k, g_ref, out_ref):
    g = g_ref[...]
    b = jax.lax.bitcast_convert_type(g, jnp.uint32)
    u = jnp.where(b >= jnp.uint32(0x80000000), ~b, b | jnp.uint32(0x80000000))
    rows = g.shape[0]
    thr = jnp.zeros((rows, 1), jnp.uint32)
    for bit in range(31, -1, -1):
        cand = thr | jnp.uint32(1 << bit)
        cnt = jnp.sum((u < cand).astype(jnp.int32), axis=1, keepdims=True)
        thr = jnp.where(cnt < k, cand, thr)
    out_ref[...] = jnp.where(u <= thr, jnp.float32(0.0), g)


def kernel(g):
    B, N = g.shape
    k = int(N * 0.5)
    import functools
    return pl.pallas_call(
        functools.partial(_ktakes_kernel, k),
        out_shape=jax.ShapeDtypeStruct((B, N), g.dtype),
    )(g)
